# Initial kernel scaffold; baseline (speedup 1.0000x reference)
import jax
import jax.numpy as jnp
from jax.experimental import pallas as pl


def kernel(indices, table, W, b):
    raise NotImplementedError("write your pallas kernel here")


# speedup vs baseline: 96.4494x; 96.4494x over previous
import functools

import jax
import jax.numpy as jnp
from jax import lax
from jax.experimental import pallas as pl
from jax.experimental.pallas import tpu as pltpu
from jax.experimental.pallas import tpu_sc as plsc

N_VOCAB = 8
DIM = 10
LANES = 16
NUM_WORKERS = 32
CHUNK = 12800


def _sc_lookup_kernel(n_total):
    per_w = n_total // NUM_WORKERS
    n_chunks = per_w // CHUNK
    mesh = plsc.VectorSubcoreMesh(core_axis_name="c", subcore_axis_name="s")

    @functools.partial(
        pl.kernel,
        out_type=jax.ShapeDtypeStruct((n_total,), jnp.float32),
        mesh=mesh,
        scratch_types=[
            pltpu.VMEM((DIM, LANES), jnp.float32),
            pltpu.VMEM((DIM, LANES), jnp.float32),
            pltpu.VMEM((LANES,), jnp.float32),
            pltpu.VMEM((CHUNK,), jnp.int32),
            pltpu.VMEM((CHUNK,), jnp.float32),
        ],
    )
    def body(idx_hbm, tab_hbm, w_hbm, b_hbm, out_hbm,
             tab_v, w_v, b_v, idx_v, out_v):
        pltpu.sync_copy(tab_hbm, tab_v)
        pltpu.sync_copy(w_hbm, w_v)
        pltpu.sync_copy(b_hbm, b_v)

        acc = b_v[...]
        for d in range(DIM):
            acc = acc + tab_v[d] * w_v[d]
        lut = 1.0 / (1.0 + jnp.exp(-acc))

        wid = lax.axis_index("s") * 2 + lax.axis_index("c")
        base = wid * per_w

        def chunk_body(c, carry):
            off = base + c * CHUNK
            pltpu.sync_copy(idx_hbm.at[pl.ds(off, CHUNK)], idx_v)

            def vec_body(i, carry2):
                iv = idx_v[pl.ds(i * LANES, LANES)]
                out_v[pl.ds(i * LANES, LANES)] = jnp.take_along_axis(
                    lut, iv, axis=0)
                return carry2

            lax.fori_loop(0, CHUNK // LANES, vec_body, 0)
            pltpu.sync_copy(out_v, out_hbm.at[pl.ds(off, CHUNK)])
            return carry

        lax.fori_loop(0, n_chunks, chunk_body, 0)

    return body


def kernel(indices, table, W, b):
    B, L = indices.shape
    n_total = B * L
    idx_flat = indices.reshape(n_total).astype(jnp.int32)

    tab_pad = jnp.zeros((DIM, LANES), jnp.float32).at[:, :N_VOCAB].set(
        table.astype(jnp.float32).T)
    w_pad = jnp.broadcast_to(
        W.reshape(DIM, 1).astype(jnp.float32), (DIM, LANES))
    b_pad = jnp.broadcast_to(b.astype(jnp.float32), (LANES,))

    out = _sc_lookup_kernel(n_total)(idx_flat, tab_pad, w_pad, b_pad)
    return out.reshape(B, L, 1)


# speedup vs baseline: 114.8410x; 1.1907x over previous
import functools

import jax
import jax.numpy as jnp
from jax import lax
from jax.experimental import pallas as pl
from jax.experimental.pallas import tpu as pltpu
from jax.experimental.pallas import tpu_sc as plsc

N_VOCAB = 8
DIM = 10
LANES = 16
NUM_WORKERS = 32
CHUNK = 12800
UNROLL = 8


def _sc_lookup_kernel(n_total):
    per_w = n_total // NUM_WORKERS
    n_chunks = per_w // CHUNK
    mesh = plsc.VectorSubcoreMesh(core_axis_name="c", subcore_axis_name="s")

    @functools.partial(
        pl.kernel,
        out_type=jax.ShapeDtypeStruct((n_total,), jnp.float32),
        mesh=mesh,
        scratch_types=[
            pltpu.VMEM((DIM, LANES), jnp.float32),
            pltpu.VMEM((DIM, LANES), jnp.float32),
            pltpu.VMEM((LANES,), jnp.float32),
            pltpu.VMEM((2, CHUNK), jnp.int32),
            pltpu.VMEM((2, CHUNK), jnp.float32),
            pltpu.SemaphoreType.DMA,
            pltpu.SemaphoreType.DMA,
            pltpu.SemaphoreType.DMA,
            pltpu.SemaphoreType.DMA,
        ],
    )
    def body(idx_hbm, tab_hbm, w_hbm, b_hbm, out_hbm,
             tab_v, w_v, b_v, idx_v, out_v,
             sem_in0, sem_in1, sem_out0, sem_out1):
        sem_in = (sem_in0, sem_in1)
        sem_out = (sem_out0, sem_out1)

        pltpu.sync_copy(tab_hbm, tab_v)
        pltpu.sync_copy(w_hbm, w_v)
        pltpu.sync_copy(b_hbm, b_v)

        acc = b_v[...]
        for d in range(DIM):
            acc = acc + tab_v[d] * w_v[d]
        lut = 1.0 / (1.0 + jnp.exp(-acc))

        wid = lax.axis_index("s") * 2 + lax.axis_index("c")
        base = wid * per_w

        def start_in(c):
            b = c & 1
            off = base + c * CHUNK
            return pltpu.async_copy(
                idx_hbm.at[pl.ds(off, CHUNK)], idx_v.at[b], sem_in[b])

        in_handles = [None, None]
        out_handles = [None, None]
        in_handles[0] = start_in(0)

        for c in range(n_chunks):
            b = c & 1
            if c + 1 < n_chunks:
                in_handles[1 - b] = start_in(c + 1)
            in_handles[b].wait()
            if out_handles[b] is not None:
                out_handles[b].wait()

            def vec_body(i, carry, b=b):
                s0 = i * (LANES * UNROLL)
                for u in range(UNROLL):
                    s = s0 + u * LANES
                    iv = idx_v[b, pl.ds(s, LANES)]
                    out_v[b, pl.ds(s, LANES)] = jnp.take_along_axis(
                        lut, iv, axis=0)
                return carry

            lax.fori_loop(0, CHUNK // (LANES * UNROLL), vec_body, 0)

            off = base + c * CHUNK
            out_handles[b] = pltpu.async_copy(
                out_v.at[b], out_hbm.at[pl.ds(off, CHUNK)], sem_out[b])

        for h in out_handles:
            if h is not None:
                h.wait()

    return body


def kernel(indices, table, W, b):
    B, L = indices.shape
    n_total = B * L
    idx_flat = indices.reshape(n_total).astype(jnp.int32)

    tab_pad = jnp.zeros((DIM, LANES), jnp.float32).at[:, :N_VOCAB].set(
        table.astype(jnp.float32).T)
    w_pad = jnp.broadcast_to(
        W.reshape(DIM, 1).astype(jnp.float32), (DIM, LANES))
    b_pad = jnp.broadcast_to(b.astype(jnp.float32), (LANES,))

    out = _sc_lookup_kernel(n_total)(idx_flat, tab_pad, w_pad, b_pad)
    return out.reshape(B, L, 1)


# speedup vs baseline: 115.6190x; 1.0068x over previous
import functools

import jax
import jax.numpy as jnp
from jax import lax
from jax.experimental import pallas as pl
from jax.experimental.pallas import tpu as pltpu
from jax.experimental.pallas import tpu_sc as plsc

N_VOCAB = 8
DIM = 10
LANES = 16
NUM_WORKERS = 32
CHUNK = 12800
UNROLL = 8


def _sc_lookup_kernel(n_total):
    per_w = n_total // NUM_WORKERS
    n_chunks = per_w // CHUNK
    mesh = plsc.VectorSubcoreMesh(core_axis_name="c", subcore_axis_name="s")

    @functools.partial(
        pl.kernel,
        out_type=jax.ShapeDtypeStruct((n_total,), jnp.float32),
        mesh=mesh,
        scratch_types=[
            pltpu.VMEM((DIM, LANES), jnp.float32),
            pltpu.VMEM((DIM, LANES), jnp.float32),
            pltpu.VMEM((LANES,), jnp.float32),
            pltpu.VMEM((2, CHUNK), jnp.int32),
            pltpu.VMEM((2, CHUNK), jnp.float32),
            pltpu.SemaphoreType.DMA,
            pltpu.SemaphoreType.DMA,
            pltpu.SemaphoreType.DMA,
            pltpu.SemaphoreType.DMA,
        ],
    )
    def body(idx_hbm, tab_hbm, w_hbm, b_hbm, out_hbm,
             tab_v, w_v, b_v, idx_v, out_v,
             sem_in0, sem_in1, sem_out0, sem_out1):
        sem_in = (sem_in0, sem_in1)
        sem_out = (sem_out0, sem_out1)

        pltpu.sync_copy(tab_hbm, tab_v)
        pltpu.sync_copy(w_hbm, w_v)
        pltpu.sync_copy(b_hbm, b_v)

        acc = b_v[...]
        for d in range(DIM):
            acc = acc + tab_v[d] * w_v[d]
        lut = 1.0 / (1.0 + jnp.exp(-acc))

        wid = lax.axis_index("s") * 2 + lax.axis_index("c")
        base = wid * per_w

        def start_in(c):
            b = c & 1
            off = base + c * CHUNK
            return pltpu.async_copy(
                idx_hbm.at[pl.ds(off, CHUNK)], idx_v.at[b], sem_in[b])

        in_handles = [None, None]
        out_handles = [None, None]
        in_handles[0] = start_in(0)

        for c in range(n_chunks):
            b = c & 1
            if c + 1 < n_chunks:
                in_handles[1 - b] = start_in(c + 1)
            in_handles[b].wait()
            if out_handles[b] is not None:
                out_handles[b].wait()

            @plsc.parallel_loop(0, CHUNK, step=LANES, unroll=UNROLL)
            def vec_body(s, b=b):
                iv = idx_v[b, pl.ds(s, LANES)]
                out_v[b, pl.ds(s, LANES)] = jnp.take_along_axis(
                    lut, iv, axis=0)

            off = base + c * CHUNK
            out_handles[b] = pltpu.async_copy(
                out_v.at[b], out_hbm.at[pl.ds(off, CHUNK)], sem_out[b])

        for h in out_handles:
            if h is not None:
                h.wait()

    return body


def kernel(indices, table, W, b):
    B, L = indices.shape
    n_total = B * L
    idx_flat = indices.reshape(n_total).astype(jnp.int32)

    tab_pad = jnp.zeros((DIM, LANES), jnp.float32).at[:, :N_VOCAB].set(
        table.astype(jnp.float32).T)
    w_pad = jnp.broadcast_to(
        W.reshape(DIM, 1).astype(jnp.float32), (DIM, LANES))
    b_pad = jnp.broadcast_to(b.astype(jnp.float32), (LANES,))

    out = _sc_lookup_kernel(n_total)(idx_flat, tab_pad, w_pad, b_pad)
    return out.reshape(B, L, 1)


# speedup vs baseline: 195.7393x; 1.6930x over previous
import functools

import jax
import jax.numpy as jnp
from jax import lax
from jax.experimental import pallas as pl
from jax.experimental.pallas import tpu as pltpu
from jax.experimental.pallas import tpu_sc as plsc

N_VOCAB = 8
DIM = 10
LANES = 16
NUM_WORKERS = 32
ROWS_PER_CHUNK = 64


def _col_offsets(n_cols):
    offs = list(range(0, n_cols - LANES + 1, LANES))
    if offs[-1] + LANES < n_cols:
        offs.append(n_cols - LANES)
    return offs


def _sc_lookup_kernel(n_rows, n_cols):
    rows_per_w = n_rows // NUM_WORKERS
    n_chunks = rows_per_w // ROWS_PER_CHUNK
    col_offs = _col_offsets(n_cols)
    mesh = plsc.VectorSubcoreMesh(core_axis_name="c", subcore_axis_name="s")

    @functools.partial(
        pl.kernel,
        out_type=jax.ShapeDtypeStruct((n_rows, n_cols), jnp.float32),
        mesh=mesh,
        scratch_types=[
            pltpu.VMEM((DIM, LANES), jnp.float32),
            pltpu.VMEM((DIM, LANES), jnp.float32),
            pltpu.VMEM((LANES,), jnp.float32),
            pltpu.VMEM((2, ROWS_PER_CHUNK, n_cols), jnp.int32),
            pltpu.VMEM((2, ROWS_PER_CHUNK, n_cols), jnp.float32),
            pltpu.SemaphoreType.DMA,
            pltpu.SemaphoreType.DMA,
            pltpu.SemaphoreType.DMA,
            pltpu.SemaphoreType.DMA,
        ],
    )
    def body(idx_hbm, tab_hbm, w_hbm, b_hbm, out_hbm,
             tab_v, w_v, b_v, idx_v, out_v,
             sem_in0, sem_in1, sem_out0, sem_out1):
        sem_in = (sem_in0, sem_in1)
        sem_out = (sem_out0, sem_out1)

        pltpu.sync_copy(tab_hbm, tab_v)
        pltpu.sync_copy(w_hbm, w_v)
        pltpu.sync_copy(b_hbm, b_v)

        acc = b_v[...]
        for d in range(DIM):
            acc = acc + tab_v[d] * w_v[d]
        lut = 1.0 / (1.0 + jnp.exp(-acc))

        wid = lax.axis_index("s") * 2 + lax.axis_index("c")
        row0 = wid * rows_per_w

        def start_in(c):
            b = c & 1
            r = row0 + c * ROWS_PER_CHUNK
            return pltpu.async_copy(
                idx_hbm.at[pl.ds(r, ROWS_PER_CHUNK), :], idx_v.at[b],
                sem_in[b])

        in_handles = [None, None]
        out_handles = [None, None]
        in_handles[0] = start_in(0)

        for c in range(n_chunks):
            b = c & 1
            if c + 1 < n_chunks:
                in_handles[1 - b] = start_in(c + 1)
            in_handles[b].wait()
            if out_handles[b] is not None:
                out_handles[b].wait()

            @plsc.parallel_loop(0, ROWS_PER_CHUNK, step=1, unroll=2)
            def row_body(r, b=b):
                for co in col_offs:
                    iv = idx_v[b, r, pl.ds(co, LANES)]
                    out_v[b, r, pl.ds(co, LANES)] = jnp.take_along_axis(
                        lut, iv, axis=0)

            r = row0 + c * ROWS_PER_CHUNK
            out_handles[b] = pltpu.async_copy(
                out_v.at[b], out_hbm.at[pl.ds(r, ROWS_PER_CHUNK), :],
                sem_out[b])

        for h in out_handles:
            if h is not None:
                h.wait()

    return body


def kernel(indices, table, W, b):
    B, L = indices.shape

    tab_pad = jnp.zeros((DIM, LANES), jnp.float32).at[:, :N_VOCAB].set(
        table.astype(jnp.float32).T)
    w_pad = jnp.broadcast_to(
        W.reshape(DIM, 1).astype(jnp.float32), (DIM, LANES))
    b_pad = jnp.broadcast_to(b.astype(jnp.float32), (LANES,))

    idx = indices.astype(jnp.int32)
    out = _sc_lookup_kernel(B, L)(idx, tab_pad, w_pad, b_pad)
    return out.reshape(B, L, 1)


# speedup vs baseline: 312.8228x; 1.5982x over previous
import functools

import jax
import jax.numpy as jnp
from jax import lax
from jax.experimental import pallas as pl
from jax.experimental.pallas import tpu as pltpu
from jax.experimental.pallas import tpu_sc as plsc

N_VOCAB = 8
DIM = 10
LANES = 16
NUM_WORKERS = 32
ROWS_PER_CHUNK = 40


def _sc_lookup_kernel(n_rows, n_cols):
    cols_per_w = n_cols // NUM_WORKERS
    n_chunks = n_rows // ROWS_PER_CHUNK
    vecs_per_row = cols_per_w // LANES
    mesh = plsc.VectorSubcoreMesh(core_axis_name="c", subcore_axis_name="s")

    @functools.partial(
        pl.kernel,
        out_type=jax.ShapeDtypeStruct((n_rows * n_cols,), jnp.float32),
        mesh=mesh,
        scratch_types=[
            pltpu.VMEM((DIM, LANES), jnp.float32),
            pltpu.VMEM((DIM, LANES), jnp.float32),
            pltpu.VMEM((LANES,), jnp.float32),
            pltpu.VMEM((2, ROWS_PER_CHUNK, cols_per_w), jnp.int32),
            pltpu.VMEM((2, ROWS_PER_CHUNK, cols_per_w), jnp.float32),
            pltpu.SemaphoreType.DMA,
            pltpu.SemaphoreType.DMA,
            pltpu.SemaphoreType.DMA,
            pltpu.SemaphoreType.DMA,
        ],
    )
    def body(idx_hbm, tab_hbm, w_hbm, b_hbm, out_hbm,
             tab_v, w_v, b_v, idx_v, out_v,
             sem_in0, sem_in1, sem_out0, sem_out1):
        sem_in = (sem_in0, sem_in1)
        sem_out = (sem_out0, sem_out1)

        pltpu.sync_copy(tab_hbm, tab_v)
        pltpu.sync_copy(w_hbm, w_v)
        pltpu.sync_copy(b_hbm, b_v)

        acc = b_v[...]
        for d in range(DIM):
            acc = acc + tab_v[d] * w_v[d]
        lut = 1.0 / (1.0 + jnp.exp(-acc))

        wid = lax.axis_index("s") * 2 + lax.axis_index("c")
        col0 = wid * cols_per_w

        def start_in(c):
            b = c & 1
            return pltpu.async_copy(
                idx_hbm.at[pl.ds(c * ROWS_PER_CHUNK, ROWS_PER_CHUNK),
                           pl.ds(col0, cols_per_w)],
                idx_v.at[b], sem_in[b])

        in_handles = [None, None]
        out_handles = [[], []]
        in_handles[0] = start_in(0)

        for c in range(n_chunks):
            b = c & 1
            if c + 1 < n_chunks:
                in_handles[1 - b] = start_in(c + 1)
            in_handles[b].wait()
            for h in out_handles[b]:
                h.wait()

            @plsc.parallel_loop(0, ROWS_PER_CHUNK, step=1)
            def row_body(r, b=b):
                for j in range(vecs_per_row):
                    iv = idx_v[b, r, pl.ds(j * LANES, LANES)]
                    out_v[b, r, pl.ds(j * LANES, LANES)] = jnp.take_along_axis(
                        lut, iv, axis=0)

            out_handles[b] = [
                pltpu.async_copy(
                    out_v.at[b, rr],
                    out_hbm.at[pl.ds(
                        (c * ROWS_PER_CHUNK + rr) * n_cols + col0,
                        cols_per_w)],
                    sem_out[b])
                for rr in range(ROWS_PER_CHUNK)
            ]

        for hs in out_handles:
            for h in hs:
                h.wait()

    return body


def kernel(indices, table, W, b):
    B, L = indices.shape

    tab_pad = jnp.zeros((DIM, LANES), jnp.float32).at[:, :N_VOCAB].set(
        table.astype(jnp.float32).T)
    w_pad = jnp.broadcast_to(
        W.reshape(DIM, 1).astype(jnp.float32), (DIM, LANES))
    b_pad = jnp.broadcast_to(b.astype(jnp.float32), (LANES,))

    idx_t = indices.astype(jnp.int32).T
    out_flat = _sc_lookup_kernel(L, B)(idx_t, tab_pad, w_pad, b_pad)
    return out_flat.reshape(L, B, 1).transpose((1, 0, 2))


# speedup vs baseline: 312.8747x; 1.0002x over previous
import functools

import jax
import jax.numpy as jnp
from jax import lax
from jax.experimental import pallas as pl
from jax.experimental.pallas import tpu as pltpu
from jax.experimental.pallas import tpu_sc as plsc

N_VOCAB = 8
DIM = 10
LANES = 16
NUM_WORKERS = 32
ROWS_PER_CHUNK = 40


def _sc_lookup_kernel(n_rows, n_cols):
    cols_per_w = n_cols // NUM_WORKERS
    n_chunks = n_rows // ROWS_PER_CHUNK
    vecs_per_row = cols_per_w // LANES
    mesh = plsc.VectorSubcoreMesh(core_axis_name="c", subcore_axis_name="s")

    @functools.partial(
        pl.kernel,
        out_type=jax.ShapeDtypeStruct((n_rows * n_cols,), jnp.float32),
        mesh=mesh,
        scratch_types=[
            pltpu.VMEM((DIM, LANES), jnp.float32),
            pltpu.VMEM((DIM, LANES), jnp.float32),
            pltpu.VMEM((LANES,), jnp.float32),
            pltpu.VMEM((2, ROWS_PER_CHUNK, cols_per_w), jnp.int32),
            pltpu.VMEM((2, ROWS_PER_CHUNK, cols_per_w), jnp.float32),
            pltpu.SemaphoreType.DMA,
            pltpu.SemaphoreType.DMA,
            pltpu.SemaphoreType.DMA,
            pltpu.SemaphoreType.DMA,
        ],
    )
    def body(idx_hbm, tab_hbm, w_hbm, b_hbm, out_hbm,
             tab_v, w_v, b_v, idx_v, out_v,
             sem_in0, sem_in1, sem_out0, sem_out1):
        sem_in = (sem_in0, sem_in1)
        sem_out = (sem_out0, sem_out1)

        pltpu.sync_copy(tab_hbm, tab_v)
        pltpu.sync_copy(w_hbm, w_v)
        pltpu.sync_copy(b_hbm, b_v)

        acc = b_v[...]
        for d in range(DIM):
            acc = acc + tab_v[d] * w_v[d]
        lut = 1.0 / (1.0 + jnp.exp(-acc))

        wid = lax.axis_index("s") * 2 + lax.axis_index("c")
        col0 = wid * cols_per_w

        def start_in(c):
            b = c & 1
            return pltpu.async_copy(
                idx_hbm.at[pl.ds(c * ROWS_PER_CHUNK, ROWS_PER_CHUNK),
                           pl.ds(col0, cols_per_w)],
                idx_v.at[b], sem_in[b])

        in_handles = [None, None]
        out_handles = [[], []]
        in_handles[0] = start_in(0)

        for c in range(n_chunks):
            b = c & 1
            if c + 1 < n_chunks:
                in_handles[1 - b] = start_in(c + 1)
            in_handles[b].wait()
            for h in out_handles[b]:
                h.wait()

            @plsc.parallel_loop(0, ROWS_PER_CHUNK, step=1)
            def row_body(r, b=b):
                for j in range(vecs_per_row):
                    iv = idx_v[b, r, pl.ds(j * LANES, LANES)]
                    out_v[b, r, pl.ds(j * LANES, LANES)] = jnp.take_along_axis(
                        lut, iv, axis=0)

            out_handles[b] = [
                pltpu.async_copy(
                    out_v.at[b, rr],
                    out_hbm.at[pl.ds(
                        (c * ROWS_PER_CHUNK + rr) * n_cols + col0,
                        cols_per_w)],
                    sem_out[b])
                for rr in range(ROWS_PER_CHUNK)
            ]

        for hs in out_handles:
            for h in hs:
                h.wait()

    return body


def kernel(indices, table, W, b):
    B, L = indices.shape

    tab_pad = jnp.pad(table.astype(jnp.float32).T,
                      ((0, 0), (0, LANES - N_VOCAB)))
    w_pad = jnp.broadcast_to(
        W.reshape(DIM, 1).astype(jnp.float32), (DIM, LANES))
    b_pad = jnp.broadcast_to(b.astype(jnp.float32), (LANES,))

    idx_t = indices.astype(jnp.int32).T
    out_flat = _sc_lookup_kernel(L, B)(idx_t, tab_pad, w_pad, b_pad)
    return out_flat.reshape(L, B, 1).transpose((1, 0, 2))


# speedup vs baseline: 349.1817x; 1.1160x over previous
import functools

import jax
import jax.numpy as jnp
from jax import lax
from jax.experimental import pallas as pl
from jax.experimental.pallas import tpu as pltpu
from jax.experimental.pallas import tpu_sc as plsc

N_VOCAB = 8
DIM = 10
LANES = 16
NUM_WORKERS = 32
ROWS_PER_CHUNK = 40


def _sc_lookup_kernel(n_rows, n_cols):
    cols_per_w = n_cols // NUM_WORKERS
    n_chunks = n_rows // ROWS_PER_CHUNK
    vecs_per_row = cols_per_w // LANES
    mesh = plsc.VectorSubcoreMesh(core_axis_name="c", subcore_axis_name="s")

    @functools.partial(
        pl.kernel,
        out_type=jax.ShapeDtypeStruct((n_rows * n_cols,), jnp.float32),
        mesh=mesh,
        scratch_types=[
            pltpu.VMEM((DIM, LANES), jnp.float32),
            pltpu.VMEM((DIM, LANES), jnp.float32),
            pltpu.VMEM((LANES,), jnp.float32),
            pltpu.VMEM((2, ROWS_PER_CHUNK, cols_per_w), jnp.int32),
            pltpu.VMEM((2, ROWS_PER_CHUNK, cols_per_w), jnp.float32),
            pltpu.SemaphoreType.DMA,
            pltpu.SemaphoreType.DMA,
            pltpu.SemaphoreType.DMA,
            pltpu.SemaphoreType.DMA,
        ],
    )
    def body(idx_hbm, tab_hbm, w_hbm, b_hbm, out_hbm,
             tab_v, w_v, b_v, idx_v, out_v,
             sem_in0, sem_in1, sem_out0, sem_out1):
        sem_in = (sem_in0, sem_in1)
        sem_out = (sem_out0, sem_out1)

        pltpu.sync_copy(tab_hbm, tab_v)
        pltpu.sync_copy(w_hbm, w_v)
        pltpu.sync_copy(b_hbm, b_v)

        acc = b_v[...]
        for d in range(DIM):
            acc = acc + tab_v[d] * w_v[d]
        lut = 1.0 / (1.0 + jnp.exp(-acc))

        wid = lax.axis_index("s") * 2 + lax.axis_index("c")
        col0 = wid * cols_per_w

        def start_in(c):
            b = c & 1
            return pltpu.async_copy(
                idx_hbm.at[pl.ds(c * ROWS_PER_CHUNK, ROWS_PER_CHUNK),
                           pl.ds(col0, cols_per_w)],
                idx_v.at[b], sem_in[b])

        def out_descr(c, rr):
            b = c & 1
            return pltpu.make_async_copy(
                out_v.at[b, rr],
                out_hbm.at[pl.ds(
                    (c * ROWS_PER_CHUNK + rr) * n_cols + col0, cols_per_w)],
                sem_out[b])

        def issue_out(c):
            def step(rr, carry):
                out_descr(c, rr).start()
                return carry
            lax.fori_loop(0, ROWS_PER_CHUNK, step, 0)

        def drain_out(c):
            def step(rr, carry):
                out_descr(c, rr).wait()
                return carry
            lax.fori_loop(0, ROWS_PER_CHUNK, step, 0)

        in_handles = [None, None]
        in_handles[0] = start_in(0)

        for c in range(n_chunks):
            b = c & 1
            if c + 1 < n_chunks:
                in_handles[1 - b] = start_in(c + 1)
            in_handles[b].wait()
            if c >= 2:
                drain_out(c - 2)

            @plsc.parallel_loop(0, ROWS_PER_CHUNK, step=1)
            def row_body(r, b=b):
                @plsc.parallel_loop(0, cols_per_w, step=LANES, unroll=4)
                def vec_body(s, r=r, b=b):
                    iv = idx_v[b, r, pl.ds(s, LANES)]
                    out_v[b, r, pl.ds(s, LANES)] = jnp.take_along_axis(
                        lut, iv, axis=0)

            issue_out(c)

        for c in range(max(0, n_chunks - 2), n_chunks):
            drain_out(c)

    return body


def kernel(indices, table, W, b):
    B, L = indices.shape

    tab_pad = jnp.pad(table.astype(jnp.float32).T,
                      ((0, 0), (0, LANES - N_VOCAB)))
    w_pad = jnp.broadcast_to(
        W.reshape(DIM, 1).astype(jnp.float32), (DIM, LANES))
    b_pad = jnp.broadcast_to(b.astype(jnp.float32), (LANES,))

    idx_t = indices.astype(jnp.int32).T
    out_flat = _sc_lookup_kernel(L, B)(idx_t, tab_pad, w_pad, b_pad)
    return out_flat.reshape(L, B, 1).transpose((1, 0, 2))


# speedup vs baseline: 374.6131x; 1.0728x over previous
import functools

import jax
import jax.numpy as jnp
from jax import lax
from jax.experimental import pallas as pl
from jax.experimental.pallas import tpu as pltpu
from jax.experimental.pallas import tpu_sc as plsc

N_VOCAB = 8
DIM = 10
LANES = 16
NUM_WORKERS = 32
ROWS_PER_CHUNK = 40


def _sc_lookup_kernel(n_rows, n_cols):
    cols_per_w = n_cols // NUM_WORKERS
    n_chunks = n_rows // ROWS_PER_CHUNK
    vecs_per_row = cols_per_w // LANES
    mesh = plsc.VectorSubcoreMesh(core_axis_name="c", subcore_axis_name="s")

    @functools.partial(
        pl.kernel,
        out_type=jax.ShapeDtypeStruct((n_rows * n_cols,), jnp.float32),
        mesh=mesh,
        scratch_types=[
            pltpu.VMEM((DIM, LANES), jnp.float32),
            pltpu.VMEM((DIM, LANES), jnp.float32),
            pltpu.VMEM((LANES,), jnp.float32),
            pltpu.VMEM((2, ROWS_PER_CHUNK, cols_per_w), jnp.int32),
            pltpu.VMEM((2, ROWS_PER_CHUNK, cols_per_w), jnp.float32),
            pltpu.SemaphoreType.DMA,
            pltpu.SemaphoreType.DMA,
            pltpu.SemaphoreType.DMA,
            pltpu.SemaphoreType.DMA,
        ],
    )
    def body(idx_hbm, tab_hbm, w_hbm, b_hbm, out_hbm,
             tab_v, w_v, b_v, idx_v, out_v,
             sem_in0, sem_in1, sem_out0, sem_out1):
        sem_in = (sem_in0, sem_in1)
        sem_out = (sem_out0, sem_out1)

        pltpu.sync_copy(tab_hbm, tab_v)
        pltpu.sync_copy(w_hbm, w_v)
        pltpu.sync_copy(b_hbm, b_v)

        acc = b_v[...]
        for d in range(DIM):
            acc = acc + tab_v[d] * w_v[d]
        lut = 1.0 / (1.0 + jnp.exp(-acc))

        wid = lax.axis_index("s") * 2 + lax.axis_index("c")
        col0 = wid * cols_per_w

        def start_in(c):
            b = c & 1
            return pltpu.async_copy(
                idx_hbm.at[pl.ds(c * ROWS_PER_CHUNK, ROWS_PER_CHUNK),
                           pl.ds(col0, cols_per_w)],
                idx_v.at[b], sem_in[b])

        def out_descr(c, rr):
            b = c & 1
            return pltpu.make_async_copy(
                out_v.at[b, rr],
                out_hbm.at[pl.ds(
                    (c * ROWS_PER_CHUNK + rr) * n_cols + col0, cols_per_w)],
                sem_out[b])

        def drain_out(c):
            def step(rr, carry):
                out_descr(c, rr).wait()
                return carry
            lax.fori_loop(0, ROWS_PER_CHUNK, step, 0)

        in_handles = [None, None]
        in_handles[0] = start_in(0)

        for c in range(n_chunks):
            b = c & 1
            if c + 1 < n_chunks:
                in_handles[1 - b] = start_in(c + 1)
            in_handles[b].wait()
            if c >= 2:
                drain_out(c - 2)

            @plsc.parallel_loop(0, ROWS_PER_CHUNK, step=1)
            def row_body(r, b=b, c=c):
                @plsc.parallel_loop(0, cols_per_w, step=LANES, unroll=8)
                def vec_body(s, r=r, b=b):
                    iv = idx_v[b, r, pl.ds(s, LANES)]
                    out_v[b, r, pl.ds(s, LANES)] = jnp.take_along_axis(
                        lut, iv, axis=0)

                out_descr(c, r).start()

        for c in range(max(0, n_chunks - 2), n_chunks):
            drain_out(c)

    return body


def kernel(indices, table, W, b):
    B, L = indices.shape

    tab_pad = jnp.pad(table.astype(jnp.float32).T,
                      ((0, 0), (0, LANES - N_VOCAB)))
    w_pad = jnp.broadcast_to(
        W.reshape(DIM, 1).astype(jnp.float32), (DIM, LANES))
    b_pad = jnp.broadcast_to(b.astype(jnp.float32), (LANES,))

    idx_t = indices.astype(jnp.int32).T
    out_flat = _sc_lookup_kernel(L, B)(idx_t, tab_pad, w_pad, b_pad)
    return out_flat.reshape(L, B, 1).transpose((1, 0, 2))


# speedup vs baseline: 375.2301x; 1.0016x over previous
import functools

import jax
import jax.numpy as jnp
from jax import lax
from jax.experimental import pallas as pl
from jax.experimental.pallas import tpu as pltpu
from jax.experimental.pallas import tpu_sc as plsc

N_VOCAB = 8
DIM = 10
LANES = 16
NUM_WORKERS = 32
ROWS_PER_CHUNK = 40


def _sc_lookup_kernel(n_rows, n_cols):
    cols_per_w = n_cols // NUM_WORKERS
    n_chunks = n_rows // ROWS_PER_CHUNK
    vecs_per_row = cols_per_w // LANES
    mesh = plsc.VectorSubcoreMesh(core_axis_name="c", subcore_axis_name="s")

    @functools.partial(
        pl.kernel,
        out_type=jax.ShapeDtypeStruct((n_rows * n_cols,), jnp.float32),
        mesh=mesh,
        scratch_types=[
            pltpu.VMEM((DIM, LANES), jnp.float32),
            pltpu.VMEM((DIM, LANES), jnp.float32),
            pltpu.VMEM((LANES,), jnp.float32),
            pltpu.VMEM((2, ROWS_PER_CHUNK, cols_per_w), jnp.int32),
            pltpu.VMEM((2, ROWS_PER_CHUNK, cols_per_w), jnp.float32),
            pltpu.SemaphoreType.DMA,
            pltpu.SemaphoreType.DMA,
            pltpu.SemaphoreType.DMA,
            pltpu.SemaphoreType.DMA,
        ],
    )
    def body(idx_hbm, tab_hbm, w_hbm, b_hbm, out_hbm,
             tab_v, w_v, b_v, idx_v, out_v,
             sem_in0, sem_in1, sem_out0, sem_out1):
        sem_in = (sem_in0, sem_in1)
        sem_out = (sem_out0, sem_out1)

        pltpu.sync_copy(tab_hbm, tab_v)
        pltpu.sync_copy(w_hbm, w_v)
        pltpu.sync_copy(b_hbm, b_v)

        acc = b_v[...]
        for d in range(DIM):
            acc = acc + tab_v[d] * w_v[d]
        lut = 1.0 / (1.0 + jnp.exp(-acc))

        wid = lax.axis_index("s") * 2 + lax.axis_index("c")
        col0 = wid * cols_per_w

        def start_in(c):
            b = c & 1
            return pltpu.async_copy(
                idx_hbm.at[pl.ds(c * ROWS_PER_CHUNK, ROWS_PER_CHUNK),
                           pl.ds(col0, cols_per_w)],
                idx_v.at[b], sem_in[b])

        def out_descr(c, rr):
            b = c & 1
            return pltpu.make_async_copy(
                out_v.at[b, rr],
                out_hbm.at[pl.ds(
                    (c * ROWS_PER_CHUNK + rr) * n_cols + col0, cols_per_w)],
                sem_out[b])

        def drain_out(c):
            def step(rr, carry):
                out_descr(c, rr).wait()
                return carry
            lax.fori_loop(0, ROWS_PER_CHUNK, step, 0)

        in_handles = [None, None]
        in_handles[0] = start_in(0)

        for c in range(n_chunks):
            b = c & 1
            if c + 1 < n_chunks:
                in_handles[1 - b] = start_in(c + 1)
            in_handles[b].wait()
            if c >= 2:
                drain_out(c - 2)

            @plsc.parallel_loop(0, ROWS_PER_CHUNK, step=1, unroll=2)
            def row_body(r, b=b, c=c):
                @plsc.parallel_loop(0, cols_per_w, step=LANES, unroll=8)
                def vec_body(s, r=r, b=b):
                    iv = idx_v[b, r, pl.ds(s, LANES)]
                    out_v[b, r, pl.ds(s, LANES)] = jnp.take_along_axis(
                        lut, iv, axis=0)

                out_descr(c, r).start()

        for c in range(max(0, n_chunks - 2), n_chunks):
            drain_out(c)

    return body


def kernel(indices, table, W, b):
    B, L = indices.shape

    tab_pad = jnp.pad(table.astype(jnp.float32).T,
                      ((0, 0), (0, LANES - N_VOCAB)))
    w_pad = jnp.broadcast_to(
        W.reshape(DIM, 1).astype(jnp.float32), (DIM, LANES))
    b_pad = jnp.broadcast_to(b.astype(jnp.float32), (LANES,))

    idx_t = indices.astype(jnp.int32).T
    out_flat = _sc_lookup_kernel(L, B)(idx_t, tab_pad, w_pad, b_pad)
    return out_flat.reshape(L, B, 1).transpose((1, 0, 2))


# speedup vs baseline: 384.4636x; 1.0246x over previous
import functools

import jax
import jax.numpy as jnp
from jax import lax
from jax.experimental import pallas as pl
from jax.experimental.pallas import tpu as pltpu
from jax.experimental.pallas import tpu_sc as plsc

N_VOCAB = 8
DIM = 10
LANES = 16
NUM_WORKERS = 32
ROWS_PER_CHUNK = 40


def _sc_lookup_kernel(n_rows, n_cols):
    cols_per_w = n_cols // NUM_WORKERS
    n_chunks = n_rows // ROWS_PER_CHUNK
    vecs_per_row = cols_per_w // LANES
    mesh = plsc.VectorSubcoreMesh(core_axis_name="c", subcore_axis_name="s")

    @functools.partial(
        pl.kernel,
        out_type=jax.ShapeDtypeStruct((n_rows * n_cols,), jnp.float32),
        mesh=mesh,
        scratch_types=[
            pltpu.VMEM((DIM, LANES), jnp.float32),
            pltpu.VMEM((DIM, LANES), jnp.float32),
            pltpu.VMEM((LANES,), jnp.float32),
            pltpu.VMEM((2, ROWS_PER_CHUNK, cols_per_w), jnp.int32),
            pltpu.VMEM((2, ROWS_PER_CHUNK, cols_per_w), jnp.float32),
            pltpu.SemaphoreType.DMA,
            pltpu.SemaphoreType.DMA,
            pltpu.SemaphoreType.DMA,
            pltpu.SemaphoreType.DMA,
        ],
    )
    def body(idx_hbm, tab_hbm, w_hbm, b_hbm, out_hbm,
             tab_v, w_v, b_v, idx_v, out_v,
             sem_in0, sem_in1, sem_out0, sem_out1):
        sem_in = (sem_in0, sem_in1)
        sem_out = (sem_out0, sem_out1)

        wid = lax.axis_index("s") * 2 + lax.axis_index("c")
        col0 = wid * cols_per_w

        def start_in(c):
            b = c & 1
            return pltpu.async_copy(
                idx_hbm.at[pl.ds(c * ROWS_PER_CHUNK, ROWS_PER_CHUNK),
                           pl.ds(col0, cols_per_w)],
                idx_v.at[b], sem_in[b])

        in_handles = [None, None]
        in_handles[0] = start_in(0)

        pltpu.sync_copy(tab_hbm, tab_v)
        pltpu.sync_copy(w_hbm, w_v)
        pltpu.sync_copy(b_hbm, b_v)

        acc = b_v[...]
        for d in range(DIM):
            acc = acc + tab_v[d] * w_v[d]
        lut = 1.0 / (1.0 + jnp.exp(-acc))

        def out_descr(c, rr):
            b = c & 1
            return pltpu.make_async_copy(
                out_v.at[b, rr],
                out_hbm.at[pl.ds(
                    (c * ROWS_PER_CHUNK + rr) * n_cols + col0, cols_per_w)],
                sem_out[b])

        def drain_out(c):
            b = c & 1
            pltpu.make_async_copy(
                idx_hbm.at[pl.ds(0, ROWS_PER_CHUNK), pl.ds(0, cols_per_w)],
                idx_v.at[b], sem_out[b]).wait()

        for c in range(n_chunks):
            b = c & 1
            if c + 1 < n_chunks:
                in_handles[1 - b] = start_in(c + 1)
            in_handles[b].wait()
            if c >= 2:
                drain_out(c - 2)

            @plsc.parallel_loop(0, ROWS_PER_CHUNK, step=1, unroll=2)
            def row_body(r, b=b, c=c):
                @plsc.parallel_loop(0, cols_per_w, step=LANES, unroll=8)
                def vec_body(s, r=r, b=b):
                    iv = idx_v[b, r, pl.ds(s, LANES)]
                    out_v[b, r, pl.ds(s, LANES)] = jnp.take_along_axis(
                        lut, iv, axis=0)

                out_descr(c, r).start()

        for c in range(max(0, n_chunks - 2), n_chunks):
            drain_out(c)

    return body


def kernel(indices, table, W, b):
    B, L = indices.shape

    tab_pad = jnp.pad(table.astype(jnp.float32).T,
                      ((0, 0), (0, LANES - N_VOCAB)))
    w_pad = jnp.broadcast_to(
        W.reshape(DIM, 1).astype(jnp.float32), (DIM, LANES))
    b_pad = jnp.broadcast_to(b.astype(jnp.float32), (LANES,))

    idx_t = indices.astype(jnp.int32).T
    out_flat = _sc_lookup_kernel(L, B)(idx_t, tab_pad, w_pad, b_pad)
    return out_flat.reshape(L, B, 1).transpose((1, 0, 2))


# speedup vs baseline: 387.8126x; 1.0087x over previous
import functools

import jax
import jax.numpy as jnp
from jax import lax
from jax.experimental import pallas as pl
from jax.experimental.pallas import tpu as pltpu
from jax.experimental.pallas import tpu_sc as plsc

N_VOCAB = 8
DIM = 10
LANES = 16
NUM_WORKERS = 32
ROWS_PER_CHUNK = 40


def _sc_lookup_kernel(n_rows, n_cols):
    cols_per_w = n_cols // NUM_WORKERS
    n_chunks = n_rows // ROWS_PER_CHUNK
    mesh = plsc.VectorSubcoreMesh(core_axis_name="c", subcore_axis_name="s")

    @functools.partial(
        pl.kernel,
        out_type=jax.ShapeDtypeStruct((n_rows * n_cols,), jnp.float32),
        mesh=mesh,
        scratch_types=[
            pltpu.VMEM((DIM, LANES), jnp.float32),
            pltpu.VMEM((DIM, LANES), jnp.float32),
            pltpu.VMEM((LANES,), jnp.float32),
            pltpu.VMEM((2, ROWS_PER_CHUNK, cols_per_w), jnp.int32),
            pltpu.VMEM((2, ROWS_PER_CHUNK, cols_per_w), jnp.float32),
            pltpu.SemaphoreType.DMA,
            pltpu.SemaphoreType.DMA,
            pltpu.SemaphoreType.DMA,
            pltpu.SemaphoreType.DMA,
        ],
    )
    def body(idx_hbm, tab_hbm, w_hbm, b_hbm, out_hbm,
             tab_v, w_v, b_v, idx_v, out_v,
             sem_in0, sem_in1, sem_out0, sem_out1):
        sem_in = (sem_in0, sem_in1)
        sem_out = (sem_out0, sem_out1)

        wid = lax.axis_index("s") * 2 + lax.axis_index("c")
        col0 = wid * cols_per_w

        def start_in(c):
            b = c & 1
            return pltpu.async_copy(
                idx_hbm.at[pl.ds(c * ROWS_PER_CHUNK, ROWS_PER_CHUNK),
                           pl.ds(col0, cols_per_w)],
                idx_v.at[b], sem_in[b])

        in_handles = [None, None]
        in_handles[0] = start_in(0)

        pltpu.sync_copy(tab_hbm, tab_v)
        pltpu.sync_copy(w_hbm, w_v)
        pltpu.sync_copy(b_hbm, b_v)

        acc = b_v[...]
        for d in range(DIM):
            acc = acc + tab_v[d] * w_v[d]
        lut = 1.0 / (1.0 + jnp.exp(-acc))

        def out_descr(c, rr):
            b = c & 1
            return pltpu.make_async_copy(
                out_v.at[b, rr],
                out_hbm.at[pl.ds(
                    (c * ROWS_PER_CHUNK + rr) * n_cols + col0, cols_per_w)],
                sem_out[b])

        def drain_out(c):
            b = c & 1
            pltpu.make_async_copy(
                idx_hbm.at[pl.ds(0, ROWS_PER_CHUNK), pl.ds(0, cols_per_w)],
                idx_v.at[b], sem_out[b]).wait()

        for c in range(n_chunks):
            b = c & 1
            if c + 1 < n_chunks:
                in_handles[1 - b] = start_in(c + 1)
            in_handles[b].wait()
            if c >= 2:
                drain_out(c - 2)

            @plsc.parallel_loop(0, ROWS_PER_CHUNK, step=1, unroll=2)
            def row_body(r, b=b, c=c):
                @plsc.parallel_loop(0, cols_per_w, step=LANES, unroll=8)
                def vec_body(s, r=r, b=b):
                    iv = idx_v[b, r, pl.ds(s, LANES)]
                    out_v[b, r, pl.ds(s, LANES)] = jnp.take_along_axis(
                        lut, iv, axis=0)

                out_descr(c, r).start()

        for c in range(max(0, n_chunks - 2), n_chunks):
            drain_out(c)

    return body


def kernel(indices, table, W, b):
    B, L = indices.shape

    tab_pad = jnp.pad(table.astype(jnp.float32).T,
                      ((0, 0), (0, LANES - N_VOCAB)))
    w_pad = jnp.broadcast_to(
        W.reshape(DIM, 1).astype(jnp.float32), (DIM, LANES))
    b_pad = jnp.broadcast_to(b.astype(jnp.float32), (LANES,))

    idx_t = indices.astype(jnp.int32).T
    out_flat = _sc_lookup_kernel(L, B)(idx_t, tab_pad, w_pad, b_pad)
    return out_flat.reshape(L, B, 1).transpose((1, 0, 2))


# speedup vs baseline: 388.0020x; 1.0005x over previous
import functools

import jax
import jax.numpy as jnp
from jax import lax
from jax.experimental import pallas as pl
from jax.experimental.pallas import tpu as pltpu
from jax.experimental.pallas import tpu_sc as plsc

N_VOCAB = 8
DIM = 10
LANES = 16
NUM_WORKERS = 32
ROWS_PER_CHUNK = 40


def _sc_lookup_kernel(n_rows, n_cols):
    cols_per_w = n_cols // NUM_WORKERS
    n_chunks = n_rows // ROWS_PER_CHUNK
    mesh = plsc.VectorSubcoreMesh(core_axis_name="c", subcore_axis_name="s")

    @functools.partial(
        pl.kernel,
        out_type=jax.ShapeDtypeStruct((n_rows * n_cols,), jnp.float32),
        mesh=mesh,
        scratch_types=[
            pltpu.VMEM((DIM, LANES), jnp.float32),
            pltpu.VMEM((DIM, LANES), jnp.float32),
            pltpu.VMEM((LANES,), jnp.float32),
            pltpu.VMEM((2, ROWS_PER_CHUNK, cols_per_w), jnp.int32),
            pltpu.VMEM((2, ROWS_PER_CHUNK, cols_per_w), jnp.float32),
            pltpu.SemaphoreType.DMA,
            pltpu.SemaphoreType.DMA,
            pltpu.SemaphoreType.DMA,
            pltpu.SemaphoreType.DMA,
        ],
    )
    def body(idx_hbm, tab_hbm, w_hbm, b_hbm, out_hbm,
             tab_v, w_v, b_v, idx_v, out_v,
             sem_in0, sem_in1, sem_out0, sem_out1):
        sem_in = (sem_in0, sem_in1)
        sem_out = (sem_out0, sem_out1)

        wid = lax.axis_index("c") * (NUM_WORKERS // 2) + lax.axis_index("s")
        col0 = wid * cols_per_w

        def start_in(c):
            b = c & 1
            return pltpu.async_copy(
                idx_hbm.at[pl.ds(c * ROWS_PER_CHUNK, ROWS_PER_CHUNK),
                           pl.ds(col0, cols_per_w)],
                idx_v.at[b], sem_in[b])

        in_handles = [None, None]
        in_handles[0] = start_in(0)

        pltpu.sync_copy(tab_hbm, tab_v)
        pltpu.sync_copy(w_hbm, w_v)
        pltpu.sync_copy(b_hbm, b_v)

        acc = b_v[...]
        for d in range(DIM):
            acc = acc + tab_v[d] * w_v[d]
        lut = 1.0 / (1.0 + jnp.exp(-acc))

        def out_descr(c, rr):
            b = c & 1
            return pltpu.make_async_copy(
                out_v.at[b, rr],
                out_hbm.at[pl.ds(
                    (c * ROWS_PER_CHUNK + rr) * n_cols + col0, cols_per_w)],
                sem_out[b])

        def drain_out(c):
            b = c & 1
            pltpu.make_async_copy(
                idx_hbm.at[pl.ds(0, ROWS_PER_CHUNK), pl.ds(0, cols_per_w)],
                idx_v.at[b], sem_out[b]).wait()

        for c in range(n_chunks):
            b = c & 1
            if c + 1 < n_chunks:
                in_handles[1 - b] = start_in(c + 1)
            in_handles[b].wait()
            if c >= 2:
                drain_out(c - 2)

            @plsc.parallel_loop(0, ROWS_PER_CHUNK, step=1, unroll=2)
            def row_body(r, b=b, c=c):
                @plsc.parallel_loop(0, cols_per_w, step=LANES, unroll=8)
                def vec_body(s, r=r, b=b):
                    iv = idx_v[b, r, pl.ds(s, LANES)]
                    out_v[b, r, pl.ds(s, LANES)] = jnp.take_along_axis(
                        lut, iv, axis=0)

                out_descr(c, r).start()

        for c in range(max(0, n_chunks - 2), n_chunks):
            drain_out(c)

    return body


def kernel(indices, table, W, b):
    B, L = indices.shape

    tab_pad = jnp.pad(table.astype(jnp.float32).T,
                      ((0, 0), (0, LANES - N_VOCAB)))
    w_pad = jnp.broadcast_to(
        W.reshape(DIM, 1).astype(jnp.float32), (DIM, LANES))
    b_pad = jnp.broadcast_to(b.astype(jnp.float32), (LANES,))

    idx_t = indices.astype(jnp.int32).T
    out_flat = _sc_lookup_kernel(L, B)(idx_t, tab_pad, w_pad, b_pad)
    return out_flat.reshape(L, B, 1).transpose((1, 0, 2))
